# Initial kernel scaffold; baseline (speedup 1.0000x reference)
import jax
import jax.numpy as jnp
from jax.experimental import pallas as pl


def kernel(z, edge_index, edge_type, weight):
    raise NotImplementedError("write your pallas kernel here")


# speedup vs baseline: 1.0814x; 1.0814x over previous
import functools

import jax
import jax.numpy as jnp
from jax import lax
from jax.experimental import pallas as pl
from jax.experimental.pallas import tpu as pltpu
from jax.experimental.pallas import tpu_sc as plsc

D = 128
E = 320000
NC = 2
NS = 16
L = 16
NW = NC * NS
E_PER_W = E // NW
CHUNK = 80
N_CHUNKS = E_PER_W // CHUNK
GROUPS = CHUNK // L
D_UNROLL = 8


def _sc_body(z_hbm, src_hbm, dst_hbm, et_hbm, w_hbm, out_hbm,
             idx_s, idx_d, idx_r, rows_s, rows_d, rows_r, out_v,
             sem_s, sem_d, sem_r):
    c = lax.axis_index("c")
    s = lax.axis_index("s")
    wid = s * NC + c
    base_w = wid * E_PER_W

    def chunk_body(i, carry):
        base = base_w + i * CHUNK
        pltpu.sync_copy(src_hbm.at[pl.ds(base, CHUNK)], idx_s)
        pltpu.sync_copy(dst_hbm.at[pl.ds(base, CHUNK)], idx_d)
        pltpu.sync_copy(et_hbm.at[pl.ds(base, CHUNK)], idx_r)
        cp_s = pltpu.async_copy(z_hbm.at[idx_s], rows_s, sem_s)
        cp_d = pltpu.async_copy(z_hbm.at[idx_d], rows_d, sem_d)
        cp_r = pltpu.async_copy(w_hbm.at[idx_r], rows_r, sem_r)
        cp_s.wait()
        cp_d.wait()
        cp_r.wait()
        for g in range(GROUPS):
            erow = jnp.full((L,), g * L, jnp.int32) + lax.iota(jnp.int32, L)

            def d_body(db, acc):
                for k in range(D_UNROLL):
                    dcol = jnp.full((L,), db * D_UNROLL + k, jnp.int32)
                    sv = plsc.load_gather(rows_s, [erow, dcol])
                    dv = plsc.load_gather(rows_d, [erow, dcol])
                    rv = plsc.load_gather(rows_r, [erow, dcol])
                    acc = acc + sv * dv * rv
                return acc

            acc = lax.fori_loop(0, D // D_UNROLL, d_body,
                                jnp.zeros((L,), jnp.float32))
            out_v[pl.ds(g * L, L)] = 1.0 / (1.0 + jnp.exp(-acc))
        pltpu.sync_copy(out_v, out_hbm.at[pl.ds(base, CHUNK)])
        return carry

    lax.fori_loop(0, N_CHUNKS, chunk_body, 0)


@jax.jit
def _sc_score(z, src, dst, et, weight):
    mesh = plsc.VectorSubcoreMesh(core_axis_name="c", subcore_axis_name="s",
                                  num_cores=NC, num_subcores=NS)
    f = pl.kernel(
        _sc_body,
        out_type=jax.ShapeDtypeStruct((E,), jnp.float32),
        mesh=mesh,
        scratch_types=[
            pltpu.VMEM((CHUNK,), jnp.int32),
            pltpu.VMEM((CHUNK,), jnp.int32),
            pltpu.VMEM((CHUNK,), jnp.int32),
            pltpu.VMEM((CHUNK, D), jnp.float32),
            pltpu.VMEM((CHUNK, D), jnp.float32),
            pltpu.VMEM((CHUNK, D), jnp.float32),
            pltpu.VMEM((CHUNK,), jnp.float32),
            pltpu.SemaphoreType.DMA,
            pltpu.SemaphoreType.DMA,
            pltpu.SemaphoreType.DMA,
        ],
        compiler_params=pltpu.CompilerParams(
            use_tc_tiling_on_sc=False,
            needs_layout_passes=False,
        ),
    )
    return f(z, src, dst, et, weight)


def kernel(z, edge_index, edge_type, weight):
    src = edge_index[0].astype(jnp.int32)
    dst = edge_index[1].astype(jnp.int32)
    et = edge_type.astype(jnp.int32)
    return _sc_score(z, src, dst, et, weight)
